# baseline (speedup 1.0000x reference)
import functools

import jax
import jax.numpy as jnp
from jax import lax
from jax.experimental import pallas as pl
from jax.experimental.pallas import tpu as pltpu
from jax.experimental.pallas import tpu_sc as plsc

LENGTH = 4096
NUM_AA = 20
BATCH = 16384
N_MUT = 32
TABLE = LENGTH * NUM_AA

NC, NS, LANES = 2, 16, 16
NW = NC * NS
B_PER_W = BATCH // NW
G_PER_W = B_PER_W // LANES


def _tc_prep(lt_ref, pos_t_ref, aa_t_ref, table_ref, fidx_ref):
    x = lt_ref[...]
    x = x - jnp.max(x, axis=0, keepdims=True)
    lse = jnp.log(jnp.sum(jnp.exp(x), axis=0, keepdims=True))
    table_ref[...] = (x - lse).reshape(TABLE)
    fidx_ref[...] = aa_t_ref[...] * LENGTH + pos_t_ref[...]


def _sc_body(table_hbm, fidx_hbm, out_hbm,
             table_v, fidx_v, out_v, sem_t, sem_f):
    wid = lax.axis_index("s") * NC + lax.axis_index("c")
    base = wid * B_PER_W
    cp_t = pltpu.make_async_copy(table_hbm, table_v, sem_t)
    cp_f = pltpu.make_async_copy(
        fidx_hbm.at[:, pl.ds(base, B_PER_W)], fidx_v, sem_f)
    cp_t.start()
    cp_f.start()
    cp_t.wait()
    cp_f.wait()

    def group(g, carry):
        sl = pl.ds(g * LANES, LANES)
        acc = jnp.zeros((LANES,), jnp.float32)
        for m in range(N_MUT):
            acc = acc + plsc.load_gather(table_v, [fidx_v[m, sl]])
        out_v[sl] = acc * (1.0 / N_MUT)
        return carry

    lax.fori_loop(0, G_PER_W, group, 0)
    pltpu.sync_copy(out_v, out_hbm.at[pl.ds(base, B_PER_W)])


@functools.cache
def _sc_call():
    return pl.kernel(
        _sc_body,
        out_type=jax.ShapeDtypeStruct((BATCH,), jnp.float32),
        mesh=plsc.VectorSubcoreMesh(
            core_axis_name="c", subcore_axis_name="s",
            num_cores=NC, num_subcores=NS,
        ),
        scratch_types=[
            pltpu.VMEM((TABLE,), jnp.float32),
            pltpu.VMEM((N_MUT, B_PER_W), jnp.int32),
            pltpu.VMEM((B_PER_W,), jnp.float32),
            pltpu.SemaphoreType.DMA,
            pltpu.SemaphoreType.DMA,
        ],
        compiler_params=pltpu.CompilerParams(needs_layout_passes=False),
    )


def kernel(logits, positions, aa_idx):
    table, fidx = pl.pallas_call(
        _tc_prep,
        out_shape=(
            jax.ShapeDtypeStruct((TABLE,), jnp.float32),
            jax.ShapeDtypeStruct((N_MUT, BATCH), jnp.int32),
        ),
    )(logits.T, positions.T, aa_idx.T)
    return _sc_call()(table, fidx)


# speedup vs baseline: 1.2346x; 1.2346x over previous
import functools

import jax
import jax.numpy as jnp
from jax import lax
from jax.experimental import pallas as pl
from jax.experimental.pallas import tpu as pltpu
from jax.experimental.pallas import tpu_sc as plsc

LENGTH = 4096
NUM_AA = 20
BATCH = 16384
N_MUT = 32
TABLE = LENGTH * NUM_AA

NC, NS, LANES = 2, 16, 16
NW = NC * NS
B_PER_W = BATCH // NW
G_PER_W = B_PER_W // LANES


def _tc_prep(lt_ref, table_ref):
    x = lt_ref[...]
    x = x - jnp.max(x, axis=0, keepdims=True)
    lse = jnp.log(jnp.sum(jnp.exp(x), axis=0, keepdims=True))
    table_ref[...] = (x - lse).reshape(TABLE)


CHUNK = TABLE // NS


def _sc_body(table_hbm, pos_hbm, aa_hbm, out_hbm,
             table_v, pos_v, aa_v, out_v, shared_t, sem_p, sem_a):
    sid = lax.axis_index("s")
    wid = sid * NC + lax.axis_index("c")
    base = wid * B_PER_W
    cp_p = pltpu.make_async_copy(
        pos_hbm.at[:, pl.ds(base, B_PER_W)], pos_v, sem_p)
    cp_a = pltpu.make_async_copy(
        aa_hbm.at[:, pl.ds(base, B_PER_W)], aa_v, sem_a)
    cp_p.start()
    cp_a.start()
    pltpu.sync_copy(table_hbm.at[pl.ds(sid * CHUNK, CHUNK)],
                    shared_t.at[pl.ds(sid * CHUNK, CHUNK)])
    plsc.subcore_barrier()
    pltpu.sync_copy(shared_t, table_v)
    cp_p.wait()
    cp_a.wait()

    def group(g, carry):
        sl = pl.ds(g * LANES, LANES)
        acc = jnp.zeros((LANES,), jnp.float32)
        for m in range(N_MUT):
            idx = aa_v[m, sl] * LENGTH + pos_v[m, sl]
            acc = acc + plsc.load_gather(table_v, [idx])
        out_v[sl] = acc * (1.0 / N_MUT)
        return carry

    lax.fori_loop(0, G_PER_W, group, 0)
    pltpu.sync_copy(out_v, out_hbm.at[pl.ds(base, B_PER_W)])


@functools.cache
def _sc_call():
    return pl.kernel(
        _sc_body,
        out_type=jax.ShapeDtypeStruct((BATCH,), jnp.float32),
        mesh=plsc.VectorSubcoreMesh(
            core_axis_name="c", subcore_axis_name="s",
            num_cores=NC, num_subcores=NS,
        ),
        scratch_types=[
            pltpu.VMEM((TABLE,), jnp.float32),
            pltpu.VMEM((N_MUT, B_PER_W), jnp.int32),
            pltpu.VMEM((N_MUT, B_PER_W), jnp.int32),
            pltpu.VMEM((B_PER_W,), jnp.float32),
            pltpu.VMEM_SHARED((TABLE,), jnp.float32),
            pltpu.SemaphoreType.DMA,
            pltpu.SemaphoreType.DMA,
        ],
        compiler_params=pltpu.CompilerParams(needs_layout_passes=False),
    )


def kernel(logits, positions, aa_idx):
    table = pl.pallas_call(
        _tc_prep,
        out_shape=jax.ShapeDtypeStruct((TABLE,), jnp.float32),
    )(logits.T)
    return _sc_call()(table, positions.T, aa_idx.T)
